# Initial kernel scaffold; baseline (speedup 1.0000x reference)
import jax
import jax.numpy as jnp
from jax.experimental import pallas as pl


def kernel(features, edge_index):
    raise NotImplementedError("write your pallas kernel here")


# speedup vs baseline: 20.7649x; 20.7649x over previous
import functools

import jax
import jax.numpy as jnp
from jax import lax
from jax.experimental import pallas as pl
from jax.experimental.pallas import tpu as pltpu
from jax.experimental.pallas import tpu_sc as plsc

N_NODES = 10000
D_FEAT = 128
N_EDGES = 320000

NC = 2
NS = 16
NW = NC * NS
CHUNK = 80
EDGES_PER_TILE = N_EDGES // NW
NCHUNK = EDGES_PER_TILE // CHUNK
N_PAD = 10240
RPT = N_PAD // NS
RZB = 128
HIST_W = 128

_mesh = plsc.VectorSubcoreMesh(
    core_axis_name="c", subcore_axis_name="s", num_cores=NC, num_subcores=NS)


def _fill_rows(buf, rows, width, value):
  v16 = jnp.full((16,), value, jnp.float32)

  def body(i, _):
    for g in range(width // 16):
      buf[i, pl.ds(g * 16, 16)] = v16
    return 0

  lax.fori_loop(0, rows, body, 0)


@functools.partial(
    pl.kernel,
    out_type=jax.ShapeDtypeStruct((NC, N_PAD, HIST_W), jnp.float32),
    mesh=_mesh,
    scratch_types=[
        pltpu.VMEM((NCHUNK, CHUNK), jnp.int32),
        pltpu.VMEM((CHUNK, HIST_W), jnp.float32),
        pltpu.VMEM_SHARED((N_PAD, HIST_W), jnp.float32),
    ],
)
def _hist_kernel(row_hbm, out_hbm, idx_v, ones_v, hist_sh):
  c = lax.axis_index("c")
  s = lax.axis_index("s")
  t = c * NS + s

  _fill_rows(ones_v, CHUNK, HIST_W, 0.0)
  for j in range(RPT // CHUNK):
    pltpu.sync_copy(ones_v, hist_sh.at[pl.ds(s * RPT + j * CHUNK, CHUNK)])
  _fill_rows(ones_v, CHUNK, HIST_W, 1.0)
  plsc.subcore_barrier()

  pltpu.sync_copy(row_hbm.at[t], idx_v)

  def body(j, _):
    pltpu.sync_copy(ones_v, hist_sh.at[idx_v.at[j]], add=True)
    return 0

  lax.fori_loop(0, NCHUNK, body, 0)
  plsc.subcore_barrier()

  pltpu.sync_copy(hist_sh.at[pl.ds(s * RPT, RPT)],
                  out_hbm.at[c, pl.ds(s * RPT, RPT)])


def _scale_body(wh_ref, f_ref, o_ref):
  rs = wh_ref[0] + wh_ref[1]
  d = jnp.where(rs > 0, lax.rsqrt(rs), jnp.zeros_like(rs))
  o_ref[...] = f_ref[...] * d[:, 0:1]


def _scale_features(wh, feats):
  n = feats.shape[0]
  bk = 1000
  grid = n // bk
  return pl.pallas_call(
      _scale_body,
      out_shape=jax.ShapeDtypeStruct(feats.shape, feats.dtype),
      grid=(grid,),
      in_specs=[
          pl.BlockSpec((NC, bk, HIST_W), lambda i: (0, i, 0)),
          pl.BlockSpec((bk, D_FEAT), lambda i: (i, 0)),
      ],
      out_specs=pl.BlockSpec((bk, D_FEAT), lambda i: (i, 0)),
  )(wh, feats)


@functools.partial(
    pl.kernel,
    out_type=jax.ShapeDtypeStruct((NC, N_PAD, D_FEAT), jnp.float32),
    mesh=_mesh,
    scratch_types=[
        pltpu.VMEM((NCHUNK, CHUNK), jnp.int32),
        pltpu.VMEM((NCHUNK, CHUNK), jnp.int32),
        pltpu.VMEM((CHUNK, D_FEAT), jnp.float32),
        pltpu.VMEM_SHARED((N_PAD, D_FEAT), jnp.float32),
        pltpu.SemaphoreType.DMA,
    ],
)
def _spmm_kernel(feat_hbm, row_hbm, col_hbm, out_hbm,
                 rowi_v, coli_v, gbuf, acc_sh, gsem):
  c = lax.axis_index("c")
  s = lax.axis_index("s")
  t = c * NS + s

  _fill_rows(gbuf, CHUNK, D_FEAT, 0.0)
  for j in range(RPT // CHUNK):
    pltpu.sync_copy(gbuf, acc_sh.at[pl.ds(s * RPT + j * CHUNK, CHUNK)])

  pltpu.sync_copy(row_hbm.at[t], rowi_v)
  pltpu.sync_copy(col_hbm.at[t], coli_v)
  plsc.subcore_barrier()

  def body(j, _):
    pltpu.async_copy(feat_hbm.at[coli_v.at[j]], gbuf, gsem).wait()
    pltpu.sync_copy(gbuf, acc_sh.at[rowi_v.at[j]], add=True)
    return 0

  lax.fori_loop(0, NCHUNK, body, 0)
  plsc.subcore_barrier()

  pltpu.sync_copy(acc_sh.at[pl.ds(s * RPT, RPT)],
                  out_hbm.at[c, pl.ds(s * RPT, RPT)])


def _final_body(p_ref, wh_ref, o_ref):
  rs = wh_ref[0] + wh_ref[1]
  d = jnp.where(rs > 0, lax.rsqrt(rs), jnp.zeros_like(rs))
  o_ref[...] = (p_ref[0] + p_ref[1]) * d[:, 0:1]


def _final_combine(parts, wh):
  n = N_NODES
  bk = 1000
  grid = n // bk
  return pl.pallas_call(
      _final_body,
      out_shape=jax.ShapeDtypeStruct((n, D_FEAT), jnp.float32),
      grid=(grid,),
      in_specs=[
          pl.BlockSpec((NC, bk, D_FEAT), lambda i: (0, i, 0)),
          pl.BlockSpec((NC, bk, HIST_W), lambda i: (0, i, 0)),
      ],
      out_specs=pl.BlockSpec((bk, D_FEAT), lambda i: (i, 0)),
  )(parts, wh)


def kernel(features, edge_index):
  ei = edge_index.astype(jnp.int32)
  row3 = ei[0].reshape(NW, NCHUNK, CHUNK)
  col3 = ei[1].reshape(NW, NCHUNK, CHUNK)
  wh = _hist_kernel(row3)
  fs = _scale_features(wh, features)
  parts = _spmm_kernel(fs, row3, col3)
  return _final_combine(parts, wh)
